# Initial kernel scaffold; baseline (speedup 1.0000x reference)
import jax
import jax.numpy as jnp
from jax.experimental import pallas as pl


def kernel(x, edge_index, W1, b1, W2, b2, W3, b3):
    raise NotImplementedError("write your pallas kernel here")


# speedup vs baseline: 9.9023x; 9.9023x over previous
import functools

import jax
import jax.numpy as jnp
from jax import lax
from jax.experimental import pallas as pl
from jax.experimental.pallas import tpu as pltpu
from jax.experimental.pallas import tpu_sc as plsc

N = 10000
E = 160000
D = 256
H = D // 2
NS = 16
BATCH = 128
ROWS_PER_TILE = 624
ROW_BLK = 400

_mesh = plsc.VectorSubcoreMesh(core_axis_name="core", subcore_axis_name="subcore")


def _slab(s):
    return pl.ds(s * ROWS_PER_TILE, ROWS_PER_TILE)


_TAIL = pl.ds(NS * ROWS_PER_TILE, N - NS * ROWS_PER_TILE)


_EDGES_PER_CORE = E // 2
_DEG_BATCHES = _EDGES_PER_CORE // BATCH
_DEG_BPT = -(-_DEG_BATCHES // NS)


@functools.partial(
    pl.kernel,
    out_type=[
        jax.ShapeDtypeStruct((N,), jnp.float32),
        jax.ShapeDtypeStruct((N,), jnp.float32),
    ],
    mesh=_mesh,
    scratch_types=[
        pltpu.VMEM_SHARED((N,), jnp.float32),
        pltpu.VMEM((2, BATCH), jnp.int32),
        pltpu.VMEM((BATCH,), jnp.float32),
        pltpu.VMEM((ROWS_PER_TILE + 16,), jnp.float32),
    ],
)
def _deg_kernel(ei_hbm, p0_hbm, p1_hbm, deg_sp, eibuf, ones, zbuf):
    c = lax.axis_index("core")
    s = lax.axis_index("subcore")

    @pl.loop(0, (ROWS_PER_TILE + 16) // 16)
    def _(i):
        zbuf[pl.ds(i * 16, 16)] = jnp.zeros((16,), jnp.float32)

    @pl.loop(0, BATCH // 16)
    def _(i):
        ones[pl.ds(i * 16, 16)] = jnp.ones((16,), jnp.float32)

    pltpu.sync_copy(zbuf.at[pl.ds(0, ROWS_PER_TILE)], deg_sp.at[_slab(s)])

    @pl.when(s == NS - 1)
    def _():
        pltpu.sync_copy(zbuf.at[pl.ds(0, 16)], deg_sp.at[_TAIL])

    plsc.subcore_barrier()

    @pl.loop(0, _DEG_BPT)
    def _(j):
        g = s * _DEG_BPT + j

        @pl.when(g < _DEG_BATCHES)
        def _():
            base = c * _EDGES_PER_CORE + g * BATCH
            pltpu.sync_copy(ei_hbm.at[:, pl.ds(base, BATCH)], eibuf)
            pltpu.sync_copy(ones, deg_sp.at[eibuf.at[1]], add=True)

    plsc.subcore_barrier()

    @pl.when(jnp.logical_and(s == 0, c == 0))
    def _():
        pltpu.sync_copy(deg_sp, p0_hbm)

    @pl.when(jnp.logical_and(s == 0, c == 1))
    def _():
        pltpu.sync_copy(deg_sp, p1_hbm)


_AGG_BATCHES = E // BATCH
_AGG_BPT = _AGG_BATCHES // NS
_AGG_REM = _AGG_BATCHES - _AGG_BPT * NS


@functools.partial(
    pl.kernel,
    out_type=[
        jax.ShapeDtypeStruct((N, H), jnp.float32),
        jax.ShapeDtypeStruct((N, H), jnp.float32),
    ],
    mesh=_mesh,
    scratch_types=[
        pltpu.VMEM_SHARED((N, H), jnp.float32),
        pltpu.VMEM((2, BATCH), jnp.int32),
        pltpu.VMEM((BATCH, H), jnp.float32),
    ],
)
def _agg_kernel(h0_hbm, h1_hbm, ei_hbm, a0_hbm, a1_hbm, acc, eibuf, msgs):
    c = lax.axis_index("core")
    s = lax.axis_index("subcore")

    def run(h_hbm, a_hbm):
        pltpu.sync_copy(h_hbm.at[_slab(s)], acc.at[_slab(s)])

        @pl.when(s == NS - 1)
        def _():
            pltpu.sync_copy(h_hbm.at[_TAIL], acc.at[_TAIL])

        plsc.subcore_barrier()

        def process(g):
            base = g * BATCH
            pltpu.sync_copy(ei_hbm.at[:, pl.ds(base, BATCH)], eibuf)
            pltpu.sync_copy(h_hbm.at[eibuf.at[0]], msgs)
            pltpu.sync_copy(msgs, acc.at[eibuf.at[1]], add=True)

        @pl.loop(0, _AGG_BPT)
        def _(j):
            process(s * _AGG_BPT + j)

        @pl.when(s < _AGG_REM)
        def _():
            process(NS * _AGG_BPT + s)

        plsc.subcore_barrier()

        pltpu.sync_copy(acc.at[_slab(s)], a_hbm.at[_slab(s)])

        @pl.when(s == NS - 1)
        def _():
            pltpu.sync_copy(acc.at[_TAIL], a_hbm.at[_TAIL])

    @pl.when(c == 0)
    def _():
        run(h0_hbm, a0_hbm)

    @pl.when(c == 1)
    def _():
        run(h1_hbm, a1_hbm)


def _dinv(p0, p1):
    return lax.rsqrt(1.0 + p0 + p1)


def _l1_body(x_ref, w_ref, p0_ref, p1_ref, h0_ref, h1_ref):
    dinv = _dinv(p0_ref[...], p1_ref[...])
    h = jnp.dot(x_ref[...], w_ref[...], preferred_element_type=jnp.float32)
    h = h * dinv
    h0_ref[...] = h[:, :H]
    h1_ref[...] = h[:, H:]


def _mid_body(a0_ref, a1_ref, p0_ref, p1_ref, b_ref, w_ref, h0_ref, h1_ref):
    dinv = _dinv(p0_ref[...], p1_ref[...])
    agg = jnp.concatenate([a0_ref[...], a1_ref[...]], axis=1)
    u = jnp.maximum(agg * dinv + b_ref[...], 0.0)
    h = jnp.dot(u, w_ref[...], preferred_element_type=jnp.float32)
    h = h * dinv
    h0_ref[...] = h[:, :H]
    h1_ref[...] = h[:, H:]


def _final_body(a0_ref, a1_ref, p0_ref, p1_ref, b_ref, out_ref):
    dinv = _dinv(p0_ref[...], p1_ref[...])
    agg = jnp.concatenate([a0_ref[...], a1_ref[...]], axis=1)
    out_ref[...] = agg * dinv + b_ref[...]


_row_spec = pl.BlockSpec((ROW_BLK, D), lambda i: (i, 0))
_half_spec = pl.BlockSpec((ROW_BLK, H), lambda i: (i, 0))
_p_spec = pl.BlockSpec((ROW_BLK, 1), lambda i: (i, 0))
_w_spec = pl.BlockSpec((D, D), lambda i: (0, 0))
_b_spec = pl.BlockSpec((1, D), lambda i: (0, 0))
_GRID = (N // ROW_BLK,)

_l1_call = pl.pallas_call(
    _l1_body,
    grid=_GRID,
    in_specs=[_row_spec, _w_spec, _p_spec, _p_spec],
    out_specs=[_half_spec, _half_spec],
    out_shape=[
        jax.ShapeDtypeStruct((N, H), jnp.float32),
        jax.ShapeDtypeStruct((N, H), jnp.float32),
    ],
)

_mid_call = pl.pallas_call(
    _mid_body,
    grid=_GRID,
    in_specs=[_half_spec, _half_spec, _p_spec, _p_spec, _b_spec, _w_spec],
    out_specs=[_half_spec, _half_spec],
    out_shape=[
        jax.ShapeDtypeStruct((N, H), jnp.float32),
        jax.ShapeDtypeStruct((N, H), jnp.float32),
    ],
)

_final_call = pl.pallas_call(
    _final_body,
    grid=_GRID,
    in_specs=[_half_spec, _half_spec, _p_spec, _p_spec, _b_spec],
    out_specs=_row_spec,
    out_shape=jax.ShapeDtypeStruct((N, D), jnp.float32),
)


def kernel(x, edge_index, W1, b1, W2, b2, W3, b3):
    ei = edge_index.astype(jnp.int32)
    p0, p1 = _deg_kernel(ei)
    p0 = p0.reshape(N, 1)
    p1 = p1.reshape(N, 1)

    h0, h1 = _l1_call(x, W1, p0, p1)
    a0, a1 = _agg_kernel(h0, h1, ei)

    h0, h1 = _mid_call(a0, a1, p0, p1, b1.reshape(1, D), W2)
    a0, a1 = _agg_kernel(h0, h1, ei)

    h0, h1 = _mid_call(a0, a1, p0, p1, b2.reshape(1, D), W3)
    a0, a1 = _agg_kernel(h0, h1, ei)

    return _final_call(a0, a1, p0, p1, b3.reshape(1, D))


# speedup vs baseline: 14.4598x; 1.4602x over previous
import functools

import jax
import jax.numpy as jnp
from jax import lax
from jax.experimental import pallas as pl
from jax.experimental.pallas import tpu as pltpu
from jax.experimental.pallas import tpu_sc as plsc

N = 10000
E = 160000
D = 256
H = D // 2
NS = 16
BATCH = 128
ROWS_PER_TILE = 624
ROW_BLK = 400

_mesh = plsc.VectorSubcoreMesh(core_axis_name="core", subcore_axis_name="subcore")


def _slab(s):
    return pl.ds(s * ROWS_PER_TILE, ROWS_PER_TILE)


_TAIL = pl.ds(NS * ROWS_PER_TILE, N - NS * ROWS_PER_TILE)


_EDGES_PER_CORE = E // 2
_DEG_BATCHES = _EDGES_PER_CORE // BATCH
_DEG_BPT = -(-_DEG_BATCHES // NS)


@functools.partial(
    pl.kernel,
    out_type=[
        jax.ShapeDtypeStruct((N,), jnp.float32),
        jax.ShapeDtypeStruct((N,), jnp.float32),
    ],
    mesh=_mesh,
    scratch_types=[
        pltpu.VMEM_SHARED((N,), jnp.float32),
        pltpu.VMEM((2, BATCH), jnp.int32),
        pltpu.VMEM((BATCH,), jnp.float32),
        pltpu.VMEM((ROWS_PER_TILE + 16,), jnp.float32),
    ],
)
def _deg_kernel(ei_hbm, p0_hbm, p1_hbm, deg_sp, eibuf, ones, zbuf):
    c = lax.axis_index("core")
    s = lax.axis_index("subcore")

    @pl.loop(0, (ROWS_PER_TILE + 16) // 16)
    def _(i):
        zbuf[pl.ds(i * 16, 16)] = jnp.zeros((16,), jnp.float32)

    @pl.loop(0, BATCH // 16)
    def _(i):
        ones[pl.ds(i * 16, 16)] = jnp.ones((16,), jnp.float32)

    pltpu.sync_copy(zbuf.at[pl.ds(0, ROWS_PER_TILE)], deg_sp.at[_slab(s)])

    @pl.when(s == NS - 1)
    def _():
        pltpu.sync_copy(zbuf.at[pl.ds(0, 16)], deg_sp.at[_TAIL])

    plsc.subcore_barrier()

    @pl.loop(0, _DEG_BPT)
    def _(j):
        g = s * _DEG_BPT + j

        @pl.when(g < _DEG_BATCHES)
        def _():
            base = c * _EDGES_PER_CORE + g * BATCH
            pltpu.sync_copy(ei_hbm.at[:, pl.ds(base, BATCH)], eibuf)
            pltpu.sync_copy(ones, deg_sp.at[eibuf.at[1]], add=True)

    plsc.subcore_barrier()

    @pl.when(jnp.logical_and(s == 0, c == 0))
    def _():
        pltpu.sync_copy(deg_sp, p0_hbm)

    @pl.when(jnp.logical_and(s == 0, c == 1))
    def _():
        pltpu.sync_copy(deg_sp, p1_hbm)


_AGG_BATCHES = E // BATCH
_AGG_BPT = _AGG_BATCHES // NS
_AGG_REM = _AGG_BATCHES - _AGG_BPT * NS


@functools.partial(
    pl.kernel,
    out_type=[
        jax.ShapeDtypeStruct((N, H), jnp.float32),
        jax.ShapeDtypeStruct((N, H), jnp.float32),
    ],
    mesh=_mesh,
    scratch_types=[
        pltpu.VMEM_SHARED((N, H), jnp.float32),
        pltpu.VMEM((2, 2, BATCH), jnp.int32),
        pltpu.VMEM((2, BATCH), jnp.int32),
        pltpu.VMEM((2, BATCH, H), jnp.float32),
        pltpu.SemaphoreType.DMA((2,)),
        pltpu.SemaphoreType.DMA((2,)),
        pltpu.SemaphoreType.DMA((2,)),
    ],
)
def _agg_kernel(h0_hbm, h1_hbm, ei_hbm, a0_hbm, a1_hbm, acc, eibuf, dstbuf,
                msgs, sem_e, sem_g, sem_s):
    c = lax.axis_index("core")
    s = lax.axis_index("subcore")
    nb = jnp.where(s < _AGG_REM, _AGG_BPT + 1, _AGG_BPT)

    def run(h_hbm, a_hbm):
        pltpu.sync_copy(h_hbm.at[_slab(s)], acc.at[_slab(s)])

        @pl.when(s == NS - 1)
        def _():
            pltpu.sync_copy(h_hbm.at[_TAIL], acc.at[_TAIL])

        plsc.subcore_barrier()

        def start_e(j, k):
            base = (j * NS + s) * BATCH
            pltpu.async_copy(ei_hbm.at[:, pl.ds(base, BATCH)], eibuf.at[k],
                             sem_e.at[k])

        start_e(0, 0)
        start_e(1, 1)

        def step(j, k):
            pltpu.make_async_copy(ei_hbm.at[:, pl.ds(0, BATCH)], eibuf.at[k],
                                  sem_e.at[k]).wait()

            @pl.when(j >= 2)
            def _():
                pltpu.make_async_copy(msgs.at[k], acc.at[dstbuf.at[k]],
                                      sem_s.at[k]).wait()

            gather = pltpu.async_copy(h_hbm.at[eibuf.at[k].at[0]], msgs.at[k],
                                      sem_g.at[k])
            gather.wait()
            for i in range(BATCH // 16):
                dstbuf[k, pl.ds(i * 16, 16)] = eibuf[k, 1, pl.ds(i * 16, 16)]

            @pl.when(j + 2 < nb)
            def _():
                start_e(j + 2, k)

            pltpu.async_copy(msgs.at[k], acc.at[dstbuf.at[k]], sem_s.at[k],
                             add=True)

        @pl.loop(0, nb)
        def _(j):
            @pl.when(j % 2 == 0)
            def _():
                step(j, 0)

            @pl.when(j % 2 == 1)
            def _():
                step(j, 1)

        for k in range(2):
            pltpu.make_async_copy(msgs.at[k], acc.at[dstbuf.at[k]],
                                  sem_s.at[k]).wait()

        plsc.subcore_barrier()

        pltpu.sync_copy(acc.at[_slab(s)], a_hbm.at[_slab(s)])

        @pl.when(s == NS - 1)
        def _():
            pltpu.sync_copy(acc.at[_TAIL], a_hbm.at[_TAIL])

    @pl.when(c == 0)
    def _():
        run(h0_hbm, a0_hbm)

    @pl.when(c == 1)
    def _():
        run(h1_hbm, a1_hbm)


def _dinv(p0, p1):
    return lax.rsqrt(1.0 + p0 + p1)


def _l1_body(x_ref, w_ref, p0_ref, p1_ref, h0_ref, h1_ref):
    dinv = _dinv(p0_ref[...], p1_ref[...])
    h = jnp.dot(x_ref[...], w_ref[...], preferred_element_type=jnp.float32)
    h = h * dinv
    h0_ref[...] = h[:, :H]
    h1_ref[...] = h[:, H:]


def _mid_body(a0_ref, a1_ref, p0_ref, p1_ref, b_ref, w_ref, h0_ref, h1_ref):
    dinv = _dinv(p0_ref[...], p1_ref[...])
    agg = jnp.concatenate([a0_ref[...], a1_ref[...]], axis=1)
    u = jnp.maximum(agg * dinv + b_ref[...], 0.0)
    h = jnp.dot(u, w_ref[...], preferred_element_type=jnp.float32)
    h = h * dinv
    h0_ref[...] = h[:, :H]
    h1_ref[...] = h[:, H:]


def _final_body(a0_ref, a1_ref, p0_ref, p1_ref, b_ref, out_ref):
    dinv = _dinv(p0_ref[...], p1_ref[...])
    agg = jnp.concatenate([a0_ref[...], a1_ref[...]], axis=1)
    out_ref[...] = agg * dinv + b_ref[...]


_row_spec = pl.BlockSpec((ROW_BLK, D), lambda i: (i, 0))
_half_spec = pl.BlockSpec((ROW_BLK, H), lambda i: (i, 0))
_p_spec = pl.BlockSpec((ROW_BLK, 1), lambda i: (i, 0))
_w_spec = pl.BlockSpec((D, D), lambda i: (0, 0))
_b_spec = pl.BlockSpec((1, D), lambda i: (0, 0))
_GRID = (N // ROW_BLK,)

_l1_call = pl.pallas_call(
    _l1_body,
    grid=_GRID,
    in_specs=[_row_spec, _w_spec, _p_spec, _p_spec],
    out_specs=[_half_spec, _half_spec],
    out_shape=[
        jax.ShapeDtypeStruct((N, H), jnp.float32),
        jax.ShapeDtypeStruct((N, H), jnp.float32),
    ],
)

_mid_call = pl.pallas_call(
    _mid_body,
    grid=_GRID,
    in_specs=[_half_spec, _half_spec, _p_spec, _p_spec, _b_spec, _w_spec],
    out_specs=[_half_spec, _half_spec],
    out_shape=[
        jax.ShapeDtypeStruct((N, H), jnp.float32),
        jax.ShapeDtypeStruct((N, H), jnp.float32),
    ],
)

_final_call = pl.pallas_call(
    _final_body,
    grid=_GRID,
    in_specs=[_half_spec, _half_spec, _p_spec, _p_spec, _b_spec],
    out_specs=_row_spec,
    out_shape=jax.ShapeDtypeStruct((N, D), jnp.float32),
)


def kernel(x, edge_index, W1, b1, W2, b2, W3, b3):
    ei = edge_index.astype(jnp.int32)
    p0, p1 = _deg_kernel(ei)
    p0 = p0.reshape(N, 1)
    p1 = p1.reshape(N, 1)

    h0, h1 = _l1_call(x, W1, p0, p1)
    a0, a1 = _agg_kernel(h0, h1, ei)

    h0, h1 = _mid_call(a0, a1, p0, p1, b1.reshape(1, D), W2)
    a0, a1 = _agg_kernel(h0, h1, ei)

    h0, h1 = _mid_call(a0, a1, p0, p1, b2.reshape(1, D), W3)
    a0, a1 = _agg_kernel(h0, h1, ei)

    return _final_call(a0, a1, p0, p1, b3.reshape(1, D))


# speedup vs baseline: 15.5343x; 1.0743x over previous
import functools

import jax
import jax.numpy as jnp
from jax import lax
from jax.experimental import pallas as pl
from jax.experimental.pallas import tpu as pltpu
from jax.experimental.pallas import tpu_sc as plsc

N = 10000
E = 160000
D = 256
H = D // 2
NS = 16
BATCH = 80
ROWS_PER_TILE = 624
ROW_BLK = 400

_mesh = plsc.VectorSubcoreMesh(core_axis_name="core", subcore_axis_name="subcore")


def _slab(s):
    return pl.ds(s * ROWS_PER_TILE, ROWS_PER_TILE)


_TAIL = pl.ds(NS * ROWS_PER_TILE, N - NS * ROWS_PER_TILE)


_EDGES_PER_CORE = E // 2
_DEG_BATCHES = _EDGES_PER_CORE // BATCH
_DEG_BPT = -(-_DEG_BATCHES // NS)


@functools.partial(
    pl.kernel,
    out_type=[
        jax.ShapeDtypeStruct((N,), jnp.float32),
        jax.ShapeDtypeStruct((N,), jnp.float32),
    ],
    mesh=_mesh,
    scratch_types=[
        pltpu.VMEM_SHARED((N,), jnp.float32),
        pltpu.VMEM((1, BATCH), jnp.int32),
        pltpu.VMEM((BATCH,), jnp.float32),
        pltpu.VMEM((ROWS_PER_TILE + 16,), jnp.float32),
    ],
)
def _deg_kernel(ei_hbm, p0_hbm, p1_hbm, deg_sp, dbuf, ones, zbuf):
    c = lax.axis_index("core")
    s = lax.axis_index("subcore")

    @pl.loop(0, (ROWS_PER_TILE + 16) // 16)
    def _(i):
        zbuf[pl.ds(i * 16, 16)] = jnp.zeros((16,), jnp.float32)

    @pl.loop(0, BATCH // 16)
    def _(i):
        ones[pl.ds(i * 16, 16)] = jnp.ones((16,), jnp.float32)

    pltpu.sync_copy(zbuf.at[pl.ds(0, ROWS_PER_TILE)], deg_sp.at[_slab(s)])

    @pl.when(s == NS - 1)
    def _():
        pltpu.sync_copy(zbuf.at[pl.ds(0, 16)], deg_sp.at[_TAIL])

    plsc.subcore_barrier()

    @pl.loop(0, _DEG_BPT)
    def _(j):
        g = s * _DEG_BPT + j

        @pl.when(g < _DEG_BATCHES)
        def _():
            base = c * _EDGES_PER_CORE + g * BATCH
            pltpu.sync_copy(ei_hbm.at[pl.ds(E + base, BATCH)], dbuf.at[0])
            pltpu.sync_copy(ones, deg_sp.at[dbuf.at[0]], add=True)

    plsc.subcore_barrier()

    @pl.when(jnp.logical_and(s == 0, c == 0))
    def _():
        pltpu.sync_copy(deg_sp, p0_hbm)

    @pl.when(jnp.logical_and(s == 0, c == 1))
    def _():
        pltpu.sync_copy(deg_sp, p1_hbm)


_AGG_BATCHES = E // BATCH
_AGG_BPT = _AGG_BATCHES // NS
assert _AGG_BPT * NS == _AGG_BATCHES


@functools.partial(
    pl.kernel,
    out_type=[
        jax.ShapeDtypeStruct((N, H), jnp.float32),
        jax.ShapeDtypeStruct((N, H), jnp.float32),
    ],
    mesh=_mesh,
    scratch_types=[
        pltpu.VMEM_SHARED((N, H), jnp.float32),
        pltpu.VMEM((4, BATCH), jnp.int32),
        pltpu.VMEM((4, BATCH), jnp.int32),
        pltpu.VMEM((4, BATCH), jnp.int32),
        pltpu.VMEM((4, BATCH, H), jnp.float32),
        pltpu.SemaphoreType.DMA((4,)),
        pltpu.SemaphoreType.DMA((4,)),
        pltpu.SemaphoreType.DMA((4,)),
    ],
)
def _agg_kernel(h0_hbm, h1_hbm, ei_hbm, a0_hbm, a1_hbm, acc, sbuf, lbuf,
                dstbuf, msgs, sem_e, sem_g, sem_s):
    c = lax.axis_index("core")
    s = lax.axis_index("subcore")
    nb = _AGG_BPT

    def run(h_hbm, a_hbm):
        pltpu.sync_copy(h_hbm.at[_slab(s)], acc.at[_slab(s)])

        @pl.when(s == NS - 1)
        def _():
            pltpu.sync_copy(h_hbm.at[_TAIL], acc.at[_TAIL])

        plsc.subcore_barrier()

        def start_e(j, k):
            base = (j * NS + s) * BATCH
            pltpu.async_copy(ei_hbm.at[pl.ds(base, BATCH)], sbuf.at[k],
                             sem_e.at[k])
            pltpu.async_copy(ei_hbm.at[pl.ds(E + base, BATCH)], lbuf.at[k],
                             sem_e.at[k])

        def wait_e(k):
            pltpu.make_async_copy(ei_hbm.at[pl.ds(0, BATCH)], sbuf.at[k],
                                  sem_e.at[k]).wait()
            pltpu.make_async_copy(ei_hbm.at[pl.ds(0, BATCH)], lbuf.at[k],
                                  sem_e.at[k]).wait()

        def wait_s(k):
            pltpu.make_async_copy(msgs.at[k], acc.at[dstbuf.at[k]],
                                  sem_s.at[k]).wait()

        for k in range(4):
            start_e(k, k)

        def step(j, m, mp):
            @pl.when(j < nb)
            def _():
                wait_e(m)

                @pl.when(j >= 4)
                def _():
                    wait_s(m)

                pltpu.async_copy(h_hbm.at[sbuf.at[m]], msgs.at[m],
                                 sem_g.at[m])

            @pl.when(j >= 1)
            def _():
                pltpu.make_async_copy(h_hbm.at[sbuf.at[mp]],
                                      msgs.at[mp], sem_g.at[mp]).wait()
                for i in range(BATCH // 16):
                    dstbuf[mp, pl.ds(i * 16, 16)] = lbuf[mp, pl.ds(i * 16, 16)]

                @pl.when(j + 3 < nb)
                def _():
                    start_e(j + 3, mp)

                pltpu.async_copy(msgs.at[mp], acc.at[dstbuf.at[mp]],
                                 sem_s.at[mp], add=True)

        @pl.loop(0, nb + 1)
        def _(j):
            for m in range(4):
                @pl.when(j % 4 == m)
                def _(m=m):
                    step(j, m, (m - 1) % 4)

        for k in range(4):
            wait_s(k)

        plsc.subcore_barrier()

        pltpu.sync_copy(acc.at[_slab(s)], a_hbm.at[_slab(s)])

        @pl.when(s == NS - 1)
        def _():
            pltpu.sync_copy(acc.at[_TAIL], a_hbm.at[_TAIL])

    @pl.when(c == 0)
    def _():
        run(h0_hbm, a0_hbm)

    @pl.when(c == 1)
    def _():
        run(h1_hbm, a1_hbm)


def _dinv(p0, p1):
    return lax.rsqrt(1.0 + p0 + p1)


def _l1_body(x_ref, w_ref, p0_ref, p1_ref, h0_ref, h1_ref):
    dinv = _dinv(p0_ref[...], p1_ref[...])
    h = jnp.dot(x_ref[...], w_ref[...], preferred_element_type=jnp.float32)
    h = h * dinv
    h0_ref[...] = h[:, :H]
    h1_ref[...] = h[:, H:]


def _mid_body(a0_ref, a1_ref, p0_ref, p1_ref, b_ref, w_ref, h0_ref, h1_ref):
    dinv = _dinv(p0_ref[...], p1_ref[...])
    agg = jnp.concatenate([a0_ref[...], a1_ref[...]], axis=1)
    u = jnp.maximum(agg * dinv + b_ref[...], 0.0)
    h = jnp.dot(u, w_ref[...], preferred_element_type=jnp.float32)
    h = h * dinv
    h0_ref[...] = h[:, :H]
    h1_ref[...] = h[:, H:]


def _final_body(a0_ref, a1_ref, p0_ref, p1_ref, b_ref, out_ref):
    dinv = _dinv(p0_ref[...], p1_ref[...])
    agg = jnp.concatenate([a0_ref[...], a1_ref[...]], axis=1)
    out_ref[...] = agg * dinv + b_ref[...]


_row_spec = pl.BlockSpec((ROW_BLK, D), lambda i: (i, 0))
_half_spec = pl.BlockSpec((ROW_BLK, H), lambda i: (i, 0))
_p_spec = pl.BlockSpec((ROW_BLK, 1), lambda i: (i, 0))
_w_spec = pl.BlockSpec((D, D), lambda i: (0, 0))
_b_spec = pl.BlockSpec((1, D), lambda i: (0, 0))
_GRID = (N // ROW_BLK,)

_l1_call = pl.pallas_call(
    _l1_body,
    grid=_GRID,
    in_specs=[_row_spec, _w_spec, _p_spec, _p_spec],
    out_specs=[_half_spec, _half_spec],
    out_shape=[
        jax.ShapeDtypeStruct((N, H), jnp.float32),
        jax.ShapeDtypeStruct((N, H), jnp.float32),
    ],
)

_mid_call = pl.pallas_call(
    _mid_body,
    grid=_GRID,
    in_specs=[_half_spec, _half_spec, _p_spec, _p_spec, _b_spec, _w_spec],
    out_specs=[_half_spec, _half_spec],
    out_shape=[
        jax.ShapeDtypeStruct((N, H), jnp.float32),
        jax.ShapeDtypeStruct((N, H), jnp.float32),
    ],
)

_final_call = pl.pallas_call(
    _final_body,
    grid=_GRID,
    in_specs=[_half_spec, _half_spec, _p_spec, _p_spec, _b_spec],
    out_specs=_row_spec,
    out_shape=jax.ShapeDtypeStruct((N, D), jnp.float32),
)


def kernel(x, edge_index, W1, b1, W2, b2, W3, b3):
    ei = edge_index.astype(jnp.int32).reshape(2 * E)
    p0, p1 = _deg_kernel(ei)
    p0 = p0.reshape(N, 1)
    p1 = p1.reshape(N, 1)

    h0, h1 = _l1_call(x, W1, p0, p1)
    a0, a1 = _agg_kernel(h0, h1, ei)

    h0, h1 = _mid_call(a0, a1, p0, p1, b1.reshape(1, D), W2)
    a0, a1 = _agg_kernel(h0, h1, ei)

    h0, h1 = _mid_call(a0, a1, p0, p1, b2.reshape(1, D), W3)
    a0, a1 = _agg_kernel(h0, h1, ei)

    return _final_call(a0, a1, p0, p1, b3.reshape(1, D))


# speedup vs baseline: 16.3177x; 1.0504x over previous
import functools

import jax
import jax.numpy as jnp
from jax import lax
from jax.experimental import pallas as pl
from jax.experimental.pallas import tpu as pltpu
from jax.experimental.pallas import tpu_sc as plsc

N = 10000
E = 160000
D = 256
H = D // 2
NS = 16
BATCH = 80
ROWS_PER_TILE = 624
ROW_BLK = 400

_mesh = plsc.VectorSubcoreMesh(core_axis_name="core", subcore_axis_name="subcore")


def _slab(s):
    return pl.ds(s * ROWS_PER_TILE, ROWS_PER_TILE)


_TAIL = pl.ds(NS * ROWS_PER_TILE, N - NS * ROWS_PER_TILE)


_EDGES_PER_CORE = E // 2
_DEG_EPT = _EDGES_PER_CORE // NS
_DEG_FULL = _DEG_EPT // BATCH
_DEG_TAILN = _DEG_EPT - _DEG_FULL * BATCH


@functools.partial(
    pl.kernel,
    out_type=[
        jax.ShapeDtypeStruct((N,), jnp.float32),
        jax.ShapeDtypeStruct((N,), jnp.float32),
    ],
    mesh=_mesh,
    scratch_types=[
        pltpu.VMEM_SHARED((N,), jnp.float32),
        pltpu.VMEM((4, BATCH), jnp.int32),
        pltpu.VMEM((1, _DEG_TAILN), jnp.int32),
        pltpu.VMEM((BATCH,), jnp.float32),
        pltpu.VMEM((ROWS_PER_TILE + 16,), jnp.float32),
        pltpu.SemaphoreType.DMA((4,)),
        pltpu.SemaphoreType.DMA((4,)),
    ],
)
def _deg_kernel(ei_hbm, p0_hbm, p1_hbm, deg_sp, dbuf, tbuf, ones, zbuf,
                sem_e, sem_s):
    c = lax.axis_index("core")
    s = lax.axis_index("subcore")

    @pl.loop(0, (ROWS_PER_TILE + 16) // 16)
    def _(i):
        zbuf[pl.ds(i * 16, 16)] = jnp.zeros((16,), jnp.float32)

    @pl.loop(0, BATCH // 16)
    def _(i):
        ones[pl.ds(i * 16, 16)] = jnp.ones((16,), jnp.float32)

    pltpu.sync_copy(zbuf.at[pl.ds(0, ROWS_PER_TILE)], deg_sp.at[_slab(s)])

    @pl.when(s == NS - 1)
    def _():
        pltpu.sync_copy(zbuf.at[pl.ds(0, 16)], deg_sp.at[_TAIL])

    plsc.subcore_barrier()

    tile_base = E + c * _EDGES_PER_CORE + s * _DEG_EPT

    def start_e(j, k):
        pltpu.async_copy(ei_hbm.at[pl.ds(tile_base + j * BATCH, BATCH)],
                         dbuf.at[k], sem_e.at[k])

    def wait_e(k):
        pltpu.make_async_copy(ei_hbm.at[pl.ds(0, BATCH)], dbuf.at[k],
                              sem_e.at[k]).wait()

    def wait_s(k):
        pltpu.make_async_copy(ones, deg_sp.at[dbuf.at[k]], sem_s.at[k]).wait()

    for k in range(4):
        start_e(k, k)

    @pl.loop(0, _DEG_FULL)
    def _(j):
        for m in range(4):
            @pl.when(j % 4 == m)
            def _(m=m):
                wait_e(m)

                @pl.when(j >= 4)
                def _():
                    wait_s(m)

                pltpu.async_copy(ones, deg_sp.at[dbuf.at[m]], sem_s.at[m],
                                 add=True)

                @pl.when(j + 4 < _DEG_FULL)
                def _():
                    start_e(j + 4, m)

    pltpu.sync_copy(ei_hbm.at[pl.ds(tile_base + _DEG_FULL * BATCH, _DEG_TAILN)],
                    tbuf.at[0])
    pltpu.sync_copy(ones.at[pl.ds(0, _DEG_TAILN)], deg_sp.at[tbuf.at[0]],
                    add=True)

    for k in range(4):
        wait_s(k)

    plsc.subcore_barrier()

    @pl.when(jnp.logical_and(s == 0, c == 0))
    def _():
        pltpu.sync_copy(deg_sp, p0_hbm)

    @pl.when(jnp.logical_and(s == 0, c == 1))
    def _():
        pltpu.sync_copy(deg_sp, p1_hbm)


_AGG_BATCHES = E // BATCH
_AGG_BPT = _AGG_BATCHES // NS
assert _AGG_BPT * NS == _AGG_BATCHES


@functools.partial(
    pl.kernel,
    out_type=[
        jax.ShapeDtypeStruct((N, H), jnp.float32),
        jax.ShapeDtypeStruct((N, H), jnp.float32),
    ],
    mesh=_mesh,
    scratch_types=[
        pltpu.VMEM_SHARED((N, H), jnp.float32),
        pltpu.VMEM((4, BATCH), jnp.int32),
        pltpu.VMEM((4, BATCH), jnp.int32),
        pltpu.VMEM((4, BATCH), jnp.int32),
        pltpu.VMEM((4, BATCH, H), jnp.float32),
        pltpu.SemaphoreType.DMA((4,)),
        pltpu.SemaphoreType.DMA((4,)),
        pltpu.SemaphoreType.DMA((4,)),
    ],
)
def _agg_kernel(h0_hbm, h1_hbm, ei_hbm, a0_hbm, a1_hbm, acc, sbuf, lbuf,
                dstbuf, msgs, sem_e, sem_g, sem_s):
    c = lax.axis_index("core")
    s = lax.axis_index("subcore")
    nb = _AGG_BPT

    def run(h_hbm, a_hbm):
        pltpu.sync_copy(h_hbm.at[_slab(s)], acc.at[_slab(s)])

        @pl.when(s == NS - 1)
        def _():
            pltpu.sync_copy(h_hbm.at[_TAIL], acc.at[_TAIL])

        plsc.subcore_barrier()

        def start_e(j, k):
            base = (j * NS + s) * BATCH
            pltpu.async_copy(ei_hbm.at[pl.ds(base, BATCH)], sbuf.at[k],
                             sem_e.at[k])
            pltpu.async_copy(ei_hbm.at[pl.ds(E + base, BATCH)], lbuf.at[k],
                             sem_e.at[k])

        def wait_e(k):
            pltpu.make_async_copy(ei_hbm.at[pl.ds(0, BATCH)], sbuf.at[k],
                                  sem_e.at[k]).wait()
            pltpu.make_async_copy(ei_hbm.at[pl.ds(0, BATCH)], lbuf.at[k],
                                  sem_e.at[k]).wait()

        def wait_s(k):
            pltpu.make_async_copy(msgs.at[k], acc.at[dstbuf.at[k]],
                                  sem_s.at[k]).wait()

        for k in range(4):
            start_e(k, k)

        def step(j, m, mp):
            @pl.when(j < nb)
            def _():
                wait_e(m)

                @pl.when(j >= 4)
                def _():
                    wait_s(m)

                pltpu.async_copy(h_hbm.at[sbuf.at[m]], msgs.at[m],
                                 sem_g.at[m])

            @pl.when(j >= 1)
            def _():
                pltpu.make_async_copy(h_hbm.at[sbuf.at[mp]],
                                      msgs.at[mp], sem_g.at[mp]).wait()
                for i in range(BATCH // 16):
                    dstbuf[mp, pl.ds(i * 16, 16)] = lbuf[mp, pl.ds(i * 16, 16)]

                @pl.when(j + 3 < nb)
                def _():
                    start_e(j + 3, mp)

                pltpu.async_copy(msgs.at[mp], acc.at[dstbuf.at[mp]],
                                 sem_s.at[mp], add=True)

        @pl.loop(0, nb + 1)
        def _(j):
            for m in range(4):
                @pl.when(j % 4 == m)
                def _(m=m):
                    step(j, m, (m - 1) % 4)

        for k in range(4):
            wait_s(k)

        plsc.subcore_barrier()

        pltpu.sync_copy(acc.at[_slab(s)], a_hbm.at[_slab(s)])

        @pl.when(s == NS - 1)
        def _():
            pltpu.sync_copy(acc.at[_TAIL], a_hbm.at[_TAIL])

    @pl.when(c == 0)
    def _():
        run(h0_hbm, a0_hbm)

    @pl.when(c == 1)
    def _():
        run(h1_hbm, a1_hbm)


def _dinv(p0, p1):
    return lax.rsqrt(1.0 + p0 + p1)


def _l1_body(x_ref, w_ref, p0_ref, p1_ref, h0_ref, h1_ref):
    dinv = _dinv(p0_ref[...], p1_ref[...])
    h = jnp.dot(x_ref[...], w_ref[...], preferred_element_type=jnp.float32)
    h = h * dinv
    h0_ref[...] = h[:, :H]
    h1_ref[...] = h[:, H:]


def _mid_body(a0_ref, a1_ref, p0_ref, p1_ref, b_ref, w_ref, h0_ref, h1_ref):
    dinv = _dinv(p0_ref[...], p1_ref[...])
    agg = jnp.concatenate([a0_ref[...], a1_ref[...]], axis=1)
    u = jnp.maximum(agg * dinv + b_ref[...], 0.0)
    h = jnp.dot(u, w_ref[...], preferred_element_type=jnp.float32)
    h = h * dinv
    h0_ref[...] = h[:, :H]
    h1_ref[...] = h[:, H:]


def _final_body(a0_ref, a1_ref, p0_ref, p1_ref, b_ref, out_ref):
    dinv = _dinv(p0_ref[...], p1_ref[...])
    agg = jnp.concatenate([a0_ref[...], a1_ref[...]], axis=1)
    out_ref[...] = agg * dinv + b_ref[...]


_row_spec = pl.BlockSpec((ROW_BLK, D), lambda i: (i, 0))
_half_spec = pl.BlockSpec((ROW_BLK, H), lambda i: (i, 0))
_p_spec = pl.BlockSpec((ROW_BLK, 1), lambda i: (i, 0))
_w_spec = pl.BlockSpec((D, D), lambda i: (0, 0))
_b_spec = pl.BlockSpec((1, D), lambda i: (0, 0))
_GRID = (N // ROW_BLK,)

_l1_call = pl.pallas_call(
    _l1_body,
    grid=_GRID,
    in_specs=[_row_spec, _w_spec, _p_spec, _p_spec],
    out_specs=[_half_spec, _half_spec],
    out_shape=[
        jax.ShapeDtypeStruct((N, H), jnp.float32),
        jax.ShapeDtypeStruct((N, H), jnp.float32),
    ],
)

_mid_call = pl.pallas_call(
    _mid_body,
    grid=_GRID,
    in_specs=[_half_spec, _half_spec, _p_spec, _p_spec, _b_spec, _w_spec],
    out_specs=[_half_spec, _half_spec],
    out_shape=[
        jax.ShapeDtypeStruct((N, H), jnp.float32),
        jax.ShapeDtypeStruct((N, H), jnp.float32),
    ],
)

_final_call = pl.pallas_call(
    _final_body,
    grid=_GRID,
    in_specs=[_half_spec, _half_spec, _p_spec, _p_spec, _b_spec],
    out_specs=_row_spec,
    out_shape=jax.ShapeDtypeStruct((N, D), jnp.float32),
)


def kernel(x, edge_index, W1, b1, W2, b2, W3, b3):
    ei = edge_index.astype(jnp.int32).reshape(2 * E)
    p0, p1 = _deg_kernel(ei)
    p0 = p0.reshape(N, 1)
    p1 = p1.reshape(N, 1)

    h0, h1 = _l1_call(x, W1, p0, p1)
    a0, a1 = _agg_kernel(h0, h1, ei)

    h0, h1 = _mid_call(a0, a1, p0, p1, b1.reshape(1, D), W2)
    a0, a1 = _agg_kernel(h0, h1, ei)

    h0, h1 = _mid_call(a0, a1, p0, p1, b2.reshape(1, D), W3)
    a0, a1 = _agg_kernel(h0, h1, ei)

    return _final_call(a0, a1, p0, p1, b3.reshape(1, D))


# speedup vs baseline: 17.9859x; 1.1022x over previous
import functools

import jax
import jax.numpy as jnp
from jax import lax
from jax.experimental import pallas as pl
from jax.experimental.pallas import tpu as pltpu
from jax.experimental.pallas import tpu_sc as plsc

N = 10000
E = 160000
D = 256
H = D // 2
NS = 16
BATCH = 80
ROWS_PER_TILE = 624
ROW_BLK = 400

_mesh = plsc.VectorSubcoreMesh(core_axis_name="core", subcore_axis_name="subcore")


def _slab(s):
    return pl.ds(s * ROWS_PER_TILE, ROWS_PER_TILE)


_TAIL = pl.ds(NS * ROWS_PER_TILE, N - NS * ROWS_PER_TILE)


_EDGES_PER_CORE = E // 2
_DEG_EPT = _EDGES_PER_CORE // NS
_DEG_FULL = _DEG_EPT // BATCH
_DEG_TAILN = _DEG_EPT - _DEG_FULL * BATCH


@functools.partial(
    pl.kernel,
    out_type=[
        jax.ShapeDtypeStruct((N,), jnp.float32),
        jax.ShapeDtypeStruct((N,), jnp.float32),
    ],
    mesh=_mesh,
    scratch_types=[
        pltpu.VMEM_SHARED((N,), jnp.float32),
        pltpu.VMEM((4, BATCH), jnp.int32),
        pltpu.VMEM((1, _DEG_TAILN), jnp.int32),
        pltpu.VMEM((BATCH,), jnp.float32),
        pltpu.VMEM((ROWS_PER_TILE + 16,), jnp.float32),
        pltpu.SemaphoreType.DMA((4,)),
        pltpu.SemaphoreType.DMA((4,)),
    ],
)
def _deg_kernel(ei_hbm, p0_hbm, p1_hbm, deg_sp, dbuf, tbuf, ones, zbuf,
                sem_e, sem_s):
    c = lax.axis_index("core")
    s = lax.axis_index("subcore")

    @pl.loop(0, (ROWS_PER_TILE + 16) // 16)
    def _(i):
        zbuf[pl.ds(i * 16, 16)] = jnp.zeros((16,), jnp.float32)

    @pl.loop(0, BATCH // 16)
    def _(i):
        ones[pl.ds(i * 16, 16)] = jnp.ones((16,), jnp.float32)

    pltpu.sync_copy(zbuf.at[pl.ds(0, ROWS_PER_TILE)], deg_sp.at[_slab(s)])

    @pl.when(s == NS - 1)
    def _():
        pltpu.sync_copy(zbuf.at[pl.ds(0, 16)], deg_sp.at[_TAIL])

    plsc.subcore_barrier()

    tile_base = E + c * _EDGES_PER_CORE + s * _DEG_EPT

    def start_e(j, k):
        pltpu.async_copy(ei_hbm.at[pl.ds(tile_base + j * BATCH, BATCH)],
                         dbuf.at[k], sem_e.at[k])

    def wait_e(k):
        pltpu.make_async_copy(ei_hbm.at[pl.ds(0, BATCH)], dbuf.at[k],
                              sem_e.at[k]).wait()

    def wait_s(k):
        pltpu.make_async_copy(ones, deg_sp.at[dbuf.at[k]], sem_s.at[k]).wait()

    for k in range(4):
        start_e(k, k)

    @pl.loop(0, _DEG_FULL)
    def _(j):
        for m in range(4):
            @pl.when(j % 4 == m)
            def _(m=m):
                wait_e(m)

                @pl.when(j >= 4)
                def _():
                    wait_s(m)

                pltpu.async_copy(ones, deg_sp.at[dbuf.at[m]], sem_s.at[m],
                                 add=True)

                @pl.when(j + 4 < _DEG_FULL)
                def _():
                    start_e(j + 4, m)

    pltpu.sync_copy(ei_hbm.at[pl.ds(tile_base + _DEG_FULL * BATCH, _DEG_TAILN)],
                    tbuf.at[0])
    pltpu.sync_copy(ones.at[pl.ds(0, _DEG_TAILN)], deg_sp.at[tbuf.at[0]],
                    add=True)

    for k in range(4):
        wait_s(k)

    plsc.subcore_barrier()

    @pl.when(jnp.logical_and(s == 0, c == 0))
    def _():
        pltpu.sync_copy(deg_sp, p0_hbm)

    @pl.when(jnp.logical_and(s == 0, c == 1))
    def _():
        pltpu.sync_copy(deg_sp, p1_hbm)


_AGG_BATCHES = E // BATCH
_AGG_BPT = _AGG_BATCHES // NS
assert _AGG_BPT * NS == _AGG_BATCHES


@functools.partial(
    pl.kernel,
    out_type=[
        jax.ShapeDtypeStruct((N, H), jnp.float32),
        jax.ShapeDtypeStruct((N, H), jnp.float32),
    ],
    mesh=_mesh,
    scratch_types=[
        pltpu.VMEM_SHARED((N, H), jnp.float32),
        pltpu.VMEM((4, BATCH), jnp.int32),
        pltpu.VMEM((4, BATCH), jnp.int32),
        pltpu.VMEM((4, BATCH), jnp.int32),
        pltpu.VMEM((4, BATCH, H), jnp.float32),
        pltpu.SemaphoreType.DMA((4,)),
        pltpu.SemaphoreType.DMA((4,)),
        pltpu.SemaphoreType.DMA((4,)),
    ],
)
def _agg_kernel(h0_hbm, h1_hbm, ei_hbm, a0_hbm, a1_hbm, acc, sbuf, lbuf,
                dstbuf, msgs, sem_e, sem_g, sem_s):
    c = lax.axis_index("core")
    s = lax.axis_index("subcore")
    nb = _AGG_BPT

    def run(h_hbm, a_hbm):
        pltpu.sync_copy(h_hbm.at[_slab(s)], acc.at[_slab(s)])

        @pl.when(s == NS - 1)
        def _():
            pltpu.sync_copy(h_hbm.at[_TAIL], acc.at[_TAIL])

        plsc.subcore_barrier()

        def start_e(j, k):
            base = (j * NS + s) * BATCH
            pltpu.async_copy(ei_hbm.at[pl.ds(base, BATCH)], sbuf.at[k],
                             sem_e.at[k])
            pltpu.async_copy(ei_hbm.at[pl.ds(E + base, BATCH)], lbuf.at[k],
                             sem_e.at[k])

        def wait_e(k):
            pltpu.make_async_copy(ei_hbm.at[pl.ds(0, BATCH)], sbuf.at[k],
                                  sem_e.at[k]).wait()
            pltpu.make_async_copy(ei_hbm.at[pl.ds(0, BATCH)], lbuf.at[k],
                                  sem_e.at[k]).wait()

        def wait_s(k):
            pltpu.make_async_copy(msgs.at[k], acc.at[dstbuf.at[k]],
                                  sem_s.at[k]).wait()

        for k in range(4):
            start_e(k, k)

        def step(j, m, mp):
            @pl.when(j < nb)
            def _():
                wait_e(m)

                @pl.when(j >= 4)
                def _():
                    wait_s(m)

                pltpu.async_copy(h_hbm.at[sbuf.at[m]], msgs.at[m],
                                 sem_g.at[m])

            @pl.when(j >= 2)
            def _():
                pltpu.make_async_copy(h_hbm.at[sbuf.at[mp]],
                                      msgs.at[mp], sem_g.at[mp]).wait()
                for i in range(BATCH // 16):
                    dstbuf[mp, pl.ds(i * 16, 16)] = lbuf[mp, pl.ds(i * 16, 16)]

                @pl.when(j + 2 < nb)
                def _():
                    start_e(j + 2, mp)

                pltpu.async_copy(msgs.at[mp], acc.at[dstbuf.at[mp]],
                                 sem_s.at[mp], add=True)

        @pl.loop(0, nb + 2)
        def _(j):
            for m in range(4):
                @pl.when(j % 4 == m)
                def _(m=m):
                    step(j, m, (m - 2) % 4)

        for k in range(4):
            wait_s(k)

        plsc.subcore_barrier()

        pltpu.sync_copy(acc.at[_slab(s)], a_hbm.at[_slab(s)])

        @pl.when(s == NS - 1)
        def _():
            pltpu.sync_copy(acc.at[_TAIL], a_hbm.at[_TAIL])

    @pl.when(c == 0)
    def _():
        run(h0_hbm, a0_hbm)

    @pl.when(c == 1)
    def _():
        run(h1_hbm, a1_hbm)


def _dinv(p0, p1):
    return lax.rsqrt(1.0 + p0 + p1)


def _l1_body(x_ref, w_ref, p0_ref, p1_ref, h0_ref, h1_ref):
    dinv = _dinv(p0_ref[...], p1_ref[...])
    h = jnp.dot(x_ref[...], w_ref[...], preferred_element_type=jnp.float32)
    h = h * dinv
    h0_ref[...] = h[:, :H]
    h1_ref[...] = h[:, H:]


def _mid_body(a0_ref, a1_ref, p0_ref, p1_ref, b_ref, w_ref, h0_ref, h1_ref):
    dinv = _dinv(p0_ref[...], p1_ref[...])
    agg = jnp.concatenate([a0_ref[...], a1_ref[...]], axis=1)
    u = jnp.maximum(agg * dinv + b_ref[...], 0.0)
    h = jnp.dot(u, w_ref[...], preferred_element_type=jnp.float32)
    h = h * dinv
    h0_ref[...] = h[:, :H]
    h1_ref[...] = h[:, H:]


def _final_body(a0_ref, a1_ref, p0_ref, p1_ref, b_ref, out_ref):
    dinv = _dinv(p0_ref[...], p1_ref[...])
    agg = jnp.concatenate([a0_ref[...], a1_ref[...]], axis=1)
    out_ref[...] = agg * dinv + b_ref[...]


_row_spec = pl.BlockSpec((ROW_BLK, D), lambda i: (i, 0))
_half_spec = pl.BlockSpec((ROW_BLK, H), lambda i: (i, 0))
_p_spec = pl.BlockSpec((ROW_BLK, 1), lambda i: (i, 0))
_w_spec = pl.BlockSpec((D, D), lambda i: (0, 0))
_b_spec = pl.BlockSpec((1, D), lambda i: (0, 0))
_GRID = (N // ROW_BLK,)

_l1_call = pl.pallas_call(
    _l1_body,
    grid=_GRID,
    in_specs=[_row_spec, _w_spec, _p_spec, _p_spec],
    out_specs=[_half_spec, _half_spec],
    out_shape=[
        jax.ShapeDtypeStruct((N, H), jnp.float32),
        jax.ShapeDtypeStruct((N, H), jnp.float32),
    ],
)

_mid_call = pl.pallas_call(
    _mid_body,
    grid=_GRID,
    in_specs=[_half_spec, _half_spec, _p_spec, _p_spec, _b_spec, _w_spec],
    out_specs=[_half_spec, _half_spec],
    out_shape=[
        jax.ShapeDtypeStruct((N, H), jnp.float32),
        jax.ShapeDtypeStruct((N, H), jnp.float32),
    ],
)

_final_call = pl.pallas_call(
    _final_body,
    grid=_GRID,
    in_specs=[_half_spec, _half_spec, _p_spec, _p_spec, _b_spec],
    out_specs=_row_spec,
    out_shape=jax.ShapeDtypeStruct((N, D), jnp.float32),
)


def kernel(x, edge_index, W1, b1, W2, b2, W3, b3):
    ei = edge_index.astype(jnp.int32).reshape(2 * E)
    p0, p1 = _deg_kernel(ei)
    p0 = p0.reshape(N, 1)
    p1 = p1.reshape(N, 1)

    h0, h1 = _l1_call(x, W1, p0, p1)
    a0, a1 = _agg_kernel(h0, h1, ei)

    h0, h1 = _mid_call(a0, a1, p0, p1, b1.reshape(1, D), W2)
    a0, a1 = _agg_kernel(h0, h1, ei)

    h0, h1 = _mid_call(a0, a1, p0, p1, b2.reshape(1, D), W3)
    a0, a1 = _agg_kernel(h0, h1, ei)

    return _final_call(a0, a1, p0, p1, b3.reshape(1, D))


# speedup vs baseline: 18.9411x; 1.0531x over previous
import functools

import jax
import jax.numpy as jnp
from jax import lax
from jax.experimental import pallas as pl
from jax.experimental.pallas import tpu as pltpu
from jax.experimental.pallas import tpu_sc as plsc

N = 10000
E = 160000
D = 256
H = D // 2
NS = 16
BATCH = 80
ROWS_PER_TILE = 624
ROW_BLK = 400

_mesh = plsc.VectorSubcoreMesh(core_axis_name="core", subcore_axis_name="subcore")


def _slab(s):
    return pl.ds(s * ROWS_PER_TILE, ROWS_PER_TILE)


_TAIL = pl.ds(NS * ROWS_PER_TILE, N - NS * ROWS_PER_TILE)


_EDGES_PER_CORE = E // 2
_DEG_EPT = _EDGES_PER_CORE // NS
_DEG_FULL = _DEG_EPT // BATCH
_DEG_TAILN = _DEG_EPT - _DEG_FULL * BATCH


@functools.partial(
    pl.kernel,
    out_type=[
        jax.ShapeDtypeStruct((N,), jnp.float32),
        jax.ShapeDtypeStruct((N,), jnp.float32),
    ],
    mesh=_mesh,
    scratch_types=[
        pltpu.VMEM_SHARED((N,), jnp.float32),
        pltpu.VMEM((4, BATCH), jnp.int32),
        pltpu.VMEM((1, _DEG_TAILN), jnp.int32),
        pltpu.VMEM((BATCH,), jnp.float32),
        pltpu.VMEM((ROWS_PER_TILE + 16,), jnp.float32),
        pltpu.SemaphoreType.DMA((4,)),
        pltpu.SemaphoreType.DMA((4,)),
    ],
)
def _deg_kernel(ei_hbm, p0_hbm, p1_hbm, deg_sp, dbuf, tbuf, ones, zbuf,
                sem_e, sem_s):
    c = lax.axis_index("core")
    s = lax.axis_index("subcore")

    @pl.loop(0, (ROWS_PER_TILE + 16) // 16)
    def _(i):
        zbuf[pl.ds(i * 16, 16)] = jnp.zeros((16,), jnp.float32)

    @pl.loop(0, BATCH // 16)
    def _(i):
        ones[pl.ds(i * 16, 16)] = jnp.ones((16,), jnp.float32)

    pltpu.sync_copy(zbuf.at[pl.ds(0, ROWS_PER_TILE)], deg_sp.at[_slab(s)])

    @pl.when(s == NS - 1)
    def _():
        pltpu.sync_copy(zbuf.at[pl.ds(0, 16)], deg_sp.at[_TAIL])

    plsc.subcore_barrier()

    tile_base = E + c * _EDGES_PER_CORE + s * _DEG_EPT

    def start_e(j, k):
        pltpu.async_copy(ei_hbm.at[pl.ds(tile_base + j * BATCH, BATCH)],
                         dbuf.at[k], sem_e.at[k])

    def wait_e(k):
        pltpu.make_async_copy(ei_hbm.at[pl.ds(0, BATCH)], dbuf.at[k],
                              sem_e.at[k]).wait()

    def wait_s(k):
        pltpu.make_async_copy(ones, deg_sp.at[dbuf.at[k]], sem_s.at[k]).wait()

    for k in range(4):
        start_e(k, k)

    @pl.loop(0, _DEG_FULL)
    def _(j):
        for m in range(4):
            @pl.when(j % 4 == m)
            def _(m=m):
                wait_e(m)

                @pl.when(j >= 4)
                def _():
                    wait_s(m)

                pltpu.async_copy(ones, deg_sp.at[dbuf.at[m]], sem_s.at[m],
                                 add=True)

                @pl.when(j + 4 < _DEG_FULL)
                def _():
                    start_e(j + 4, m)

    pltpu.sync_copy(ei_hbm.at[pl.ds(tile_base + _DEG_FULL * BATCH, _DEG_TAILN)],
                    tbuf.at[0])
    pltpu.sync_copy(ones.at[pl.ds(0, _DEG_TAILN)], deg_sp.at[tbuf.at[0]],
                    add=True)

    for k in range(4):
        wait_s(k)

    plsc.subcore_barrier()

    @pl.when(jnp.logical_and(s == 0, c == 0))
    def _():
        pltpu.sync_copy(deg_sp, p0_hbm)

    @pl.when(jnp.logical_and(s == 0, c == 1))
    def _():
        pltpu.sync_copy(deg_sp, p1_hbm)


_AGG_BATCHES = E // BATCH
_AGG_BPT = _AGG_BATCHES // NS
assert _AGG_BPT * NS == _AGG_BATCHES


@functools.partial(
    pl.kernel,
    out_type=[
        jax.ShapeDtypeStruct((N, H), jnp.float32),
        jax.ShapeDtypeStruct((N, H), jnp.float32),
    ],
    mesh=_mesh,
    scratch_types=[
        pltpu.VMEM_SHARED((N, H), jnp.float32),
        pltpu.VMEM((8, BATCH), jnp.int32),
        pltpu.VMEM((8, BATCH), jnp.int32),
        pltpu.VMEM((8, BATCH), jnp.int32),
        pltpu.VMEM((4, BATCH, H), jnp.float32),
        pltpu.SemaphoreType.DMA((8,)),
        pltpu.SemaphoreType.DMA((4,)),
        pltpu.SemaphoreType.DMA((4,)),
    ],
)
def _agg_kernel(h0_hbm, h1_hbm, ei_hbm, a0_hbm, a1_hbm, acc, sbuf, lbuf,
                dstbuf, msgs, sem_e, sem_g, sem_s):
    c = lax.axis_index("core")
    s = lax.axis_index("subcore")
    nb = _AGG_BPT

    def run(h_hbm, a_hbm):
        pltpu.sync_copy(h_hbm.at[_slab(s)], acc.at[_slab(s)])

        @pl.when(s == NS - 1)
        def _():
            pltpu.sync_copy(h_hbm.at[_TAIL], acc.at[_TAIL])

        plsc.subcore_barrier()

        def start_e(j, k):
            base = (j * NS + s) * BATCH
            pltpu.async_copy(ei_hbm.at[pl.ds(base, BATCH)], sbuf.at[k],
                             sem_e.at[k])
            pltpu.async_copy(ei_hbm.at[pl.ds(E + base, BATCH)], lbuf.at[k],
                             sem_e.at[k])

        def wait_e(k):
            pltpu.make_async_copy(ei_hbm.at[pl.ds(0, BATCH)], sbuf.at[k],
                                  sem_e.at[k]).wait()
            pltpu.make_async_copy(ei_hbm.at[pl.ds(0, BATCH)], lbuf.at[k],
                                  sem_e.at[k]).wait()

        def wait_s(k4, k8):
            pltpu.make_async_copy(msgs.at[k4], acc.at[dstbuf.at[k8]],
                                  sem_s.at[k4]).wait()

        for k in range(8):
            start_e(k, k)

        def step(j, m8):
            m4 = m8 % 4
            md8 = (m8 - 3) % 8
            md4 = (m8 - 3) % 4
            @pl.when(j < nb)
            def _():
                wait_e(m8)

                @pl.when(j >= 4)
                def _():
                    wait_s(m4, (m8 - 4) % 8)

                pltpu.async_copy(h_hbm.at[sbuf.at[m8]], msgs.at[m4],
                                 sem_g.at[m4])

            @pl.when(j >= 3)
            def _():
                pltpu.make_async_copy(h_hbm.at[sbuf.at[md8]],
                                      msgs.at[md4], sem_g.at[md4]).wait()
                for i in range(BATCH // 16):
                    dstbuf[md8, pl.ds(i * 16, 16)] = lbuf[md8,
                                                          pl.ds(i * 16, 16)]

                @pl.when(j + 5 < nb)
                def _():
                    start_e(j + 5, md8)

                pltpu.async_copy(msgs.at[md4], acc.at[dstbuf.at[md8]],
                                 sem_s.at[md4], add=True)

        @pl.loop(0, nb + 3)
        def _(j):
            for m in range(8):
                @pl.when(j % 8 == m)
                def _(m=m):
                    step(j, m)

        for t in range(4):
            g = nb - 4 + t
            wait_s(g % 4, g % 8)

        plsc.subcore_barrier()

        pltpu.sync_copy(acc.at[_slab(s)], a_hbm.at[_slab(s)])

        @pl.when(s == NS - 1)
        def _():
            pltpu.sync_copy(acc.at[_TAIL], a_hbm.at[_TAIL])

    @pl.when(c == 0)
    def _():
        run(h0_hbm, a0_hbm)

    @pl.when(c == 1)
    def _():
        run(h1_hbm, a1_hbm)


def _dinv(p0, p1):
    return lax.rsqrt(1.0 + p0 + p1)


def _l1_body(x_ref, w_ref, p0_ref, p1_ref, h0_ref, h1_ref):
    dinv = _dinv(p0_ref[...], p1_ref[...])
    h = jnp.dot(x_ref[...], w_ref[...], preferred_element_type=jnp.float32)
    h = h * dinv
    h0_ref[...] = h[:, :H]
    h1_ref[...] = h[:, H:]


def _mid_body(a0_ref, a1_ref, p0_ref, p1_ref, b_ref, w_ref, h0_ref, h1_ref):
    dinv = _dinv(p0_ref[...], p1_ref[...])
    agg = jnp.concatenate([a0_ref[...], a1_ref[...]], axis=1)
    u = jnp.maximum(agg * dinv + b_ref[...], 0.0)
    h = jnp.dot(u, w_ref[...], preferred_element_type=jnp.float32)
    h = h * dinv
    h0_ref[...] = h[:, :H]
    h1_ref[...] = h[:, H:]


def _final_body(a0_ref, a1_ref, p0_ref, p1_ref, b_ref, out_ref):
    dinv = _dinv(p0_ref[...], p1_ref[...])
    agg = jnp.concatenate([a0_ref[...], a1_ref[...]], axis=1)
    out_ref[...] = agg * dinv + b_ref[...]


_row_spec = pl.BlockSpec((ROW_BLK, D), lambda i: (i, 0))
_half_spec = pl.BlockSpec((ROW_BLK, H), lambda i: (i, 0))
_p_spec = pl.BlockSpec((ROW_BLK, 1), lambda i: (i, 0))
_w_spec = pl.BlockSpec((D, D), lambda i: (0, 0))
_b_spec = pl.BlockSpec((1, D), lambda i: (0, 0))
_GRID = (N // ROW_BLK,)

_l1_call = pl.pallas_call(
    _l1_body,
    grid=_GRID,
    in_specs=[_row_spec, _w_spec, _p_spec, _p_spec],
    out_specs=[_half_spec, _half_spec],
    out_shape=[
        jax.ShapeDtypeStruct((N, H), jnp.float32),
        jax.ShapeDtypeStruct((N, H), jnp.float32),
    ],
)

_mid_call = pl.pallas_call(
    _mid_body,
    grid=_GRID,
    in_specs=[_half_spec, _half_spec, _p_spec, _p_spec, _b_spec, _w_spec],
    out_specs=[_half_spec, _half_spec],
    out_shape=[
        jax.ShapeDtypeStruct((N, H), jnp.float32),
        jax.ShapeDtypeStruct((N, H), jnp.float32),
    ],
)

_final_call = pl.pallas_call(
    _final_body,
    grid=_GRID,
    in_specs=[_half_spec, _half_spec, _p_spec, _p_spec, _b_spec],
    out_specs=_row_spec,
    out_shape=jax.ShapeDtypeStruct((N, D), jnp.float32),
)


def kernel(x, edge_index, W1, b1, W2, b2, W3, b3):
    ei = edge_index.astype(jnp.int32).reshape(2 * E)
    p0, p1 = _deg_kernel(ei)
    p0 = p0.reshape(N, 1)
    p1 = p1.reshape(N, 1)

    h0, h1 = _l1_call(x, W1, p0, p1)
    a0, a1 = _agg_kernel(h0, h1, ei)

    h0, h1 = _mid_call(a0, a1, p0, p1, b1.reshape(1, D), W2)
    a0, a1 = _agg_kernel(h0, h1, ei)

    h0, h1 = _mid_call(a0, a1, p0, p1, b2.reshape(1, D), W3)
    a0, a1 = _agg_kernel(h0, h1, ei)

    return _final_call(a0, a1, p0, p1, b3.reshape(1, D))


# speedup vs baseline: 20.8757x; 1.1021x over previous
import functools

import jax
import jax.numpy as jnp
from jax import lax
from jax.experimental import pallas as pl
from jax.experimental.pallas import tpu as pltpu
from jax.experimental.pallas import tpu_sc as plsc

N = 10000
E = 160000
D = 256
H = D // 2
NS = 16
BATCH = 80
ROWS_PER_TILE = 624
ROW_BLK = 2000

_mesh = plsc.VectorSubcoreMesh(core_axis_name="core", subcore_axis_name="subcore")


def _slab(s):
    return pl.ds(s * ROWS_PER_TILE, ROWS_PER_TILE)


_TAIL = pl.ds(NS * ROWS_PER_TILE, N - NS * ROWS_PER_TILE)


_EDGES_PER_CORE = E // 2
_DEG_EPT = _EDGES_PER_CORE // NS
_DEG_FULL = _DEG_EPT // BATCH
_DEG_TAILN = _DEG_EPT - _DEG_FULL * BATCH


@functools.partial(
    pl.kernel,
    out_type=[
        jax.ShapeDtypeStruct((N,), jnp.float32),
        jax.ShapeDtypeStruct((N,), jnp.float32),
    ],
    mesh=_mesh,
    scratch_types=[
        pltpu.VMEM_SHARED((N,), jnp.float32),
        pltpu.VMEM((4, BATCH), jnp.int32),
        pltpu.VMEM((1, _DEG_TAILN), jnp.int32),
        pltpu.VMEM((BATCH,), jnp.float32),
        pltpu.VMEM((ROWS_PER_TILE + 16,), jnp.float32),
        pltpu.SemaphoreType.DMA((4,)),
        pltpu.SemaphoreType.DMA((4,)),
    ],
)
def _deg_kernel(ei_hbm, p0_hbm, p1_hbm, deg_sp, dbuf, tbuf, ones, zbuf,
                sem_e, sem_s):
    c = lax.axis_index("core")
    s = lax.axis_index("subcore")

    @pl.loop(0, (ROWS_PER_TILE + 16) // 16)
    def _(i):
        zbuf[pl.ds(i * 16, 16)] = jnp.zeros((16,), jnp.float32)

    @pl.loop(0, BATCH // 16)
    def _(i):
        ones[pl.ds(i * 16, 16)] = jnp.ones((16,), jnp.float32)

    pltpu.sync_copy(zbuf.at[pl.ds(0, ROWS_PER_TILE)], deg_sp.at[_slab(s)])

    @pl.when(s == NS - 1)
    def _():
        pltpu.sync_copy(zbuf.at[pl.ds(0, 16)], deg_sp.at[_TAIL])

    plsc.subcore_barrier()

    tile_base = E + c * _EDGES_PER_CORE + s * _DEG_EPT

    def start_e(j, k):
        pltpu.async_copy(ei_hbm.at[pl.ds(tile_base + j * BATCH, BATCH)],
                         dbuf.at[k], sem_e.at[k])

    def wait_e(k):
        pltpu.make_async_copy(ei_hbm.at[pl.ds(0, BATCH)], dbuf.at[k],
                              sem_e.at[k]).wait()

    def wait_s(k):
        pltpu.make_async_copy(ones, deg_sp.at[dbuf.at[k]], sem_s.at[k]).wait()

    for k in range(4):
        start_e(k, k)

    @pl.loop(0, _DEG_FULL)
    def _(j):
        for m in range(4):
            @pl.when(j % 4 == m)
            def _(m=m):
                wait_e(m)

                @pl.when(j >= 4)
                def _():
                    wait_s(m)

                pltpu.async_copy(ones, deg_sp.at[dbuf.at[m]], sem_s.at[m],
                                 add=True)

                @pl.when(j + 4 < _DEG_FULL)
                def _():
                    start_e(j + 4, m)

    pltpu.sync_copy(ei_hbm.at[pl.ds(tile_base + _DEG_FULL * BATCH, _DEG_TAILN)],
                    tbuf.at[0])
    pltpu.sync_copy(ones.at[pl.ds(0, _DEG_TAILN)], deg_sp.at[tbuf.at[0]],
                    add=True)

    for k in range(4):
        wait_s(k)

    plsc.subcore_barrier()

    @pl.when(jnp.logical_and(s == 0, c == 0))
    def _():
        pltpu.sync_copy(deg_sp, p0_hbm)

    @pl.when(jnp.logical_and(s == 0, c == 1))
    def _():
        pltpu.sync_copy(deg_sp, p1_hbm)


_AGG_BATCHES = E // BATCH
_AGG_BPT = _AGG_BATCHES // NS
assert _AGG_BPT * NS == _AGG_BATCHES


@functools.partial(
    pl.kernel,
    out_type=[
        jax.ShapeDtypeStruct((N, H), jnp.float32),
        jax.ShapeDtypeStruct((N, H), jnp.float32),
    ],
    mesh=_mesh,
    scratch_types=[
        pltpu.VMEM_SHARED((N, H), jnp.float32),
        pltpu.VMEM((8, BATCH), jnp.int32),
        pltpu.VMEM((8, BATCH), jnp.int32),
        pltpu.VMEM((8, BATCH), jnp.int32),
        pltpu.VMEM((4, BATCH, H), jnp.float32),
        pltpu.SemaphoreType.DMA((8,)),
        pltpu.SemaphoreType.DMA((4,)),
        pltpu.SemaphoreType.DMA((4,)),
    ],
)
def _agg_kernel(h0_hbm, h1_hbm, ei_hbm, a0_hbm, a1_hbm, acc, sbuf, lbuf,
                dstbuf, msgs, sem_e, sem_g, sem_s):
    c = lax.axis_index("core")
    s = lax.axis_index("subcore")
    nb = _AGG_BPT

    def run(h_hbm, a_hbm):
        pltpu.sync_copy(h_hbm.at[_slab(s)], acc.at[_slab(s)])

        @pl.when(s == NS - 1)
        def _():
            pltpu.sync_copy(h_hbm.at[_TAIL], acc.at[_TAIL])

        plsc.subcore_barrier()

        def start_e(j, k):
            base = (j * NS + s) * BATCH
            pltpu.async_copy(ei_hbm.at[pl.ds(base, BATCH)], sbuf.at[k],
                             sem_e.at[k])
            pltpu.async_copy(ei_hbm.at[pl.ds(E + base, BATCH)], lbuf.at[k],
                             sem_e.at[k])

        def wait_e(k):
            pltpu.make_async_copy(ei_hbm.at[pl.ds(0, BATCH)], sbuf.at[k],
                                  sem_e.at[k]).wait()
            pltpu.make_async_copy(ei_hbm.at[pl.ds(0, BATCH)], lbuf.at[k],
                                  sem_e.at[k]).wait()

        def wait_s(k4, k8):
            pltpu.make_async_copy(msgs.at[k4], acc.at[dstbuf.at[k8]],
                                  sem_s.at[k4]).wait()

        for k in range(8):
            start_e(k, k)

        def step(j, m8):
            m4 = m8 % 4
            md8 = (m8 - 3) % 8
            md4 = (m8 - 3) % 4
            @pl.when(j < nb)
            def _():
                wait_e(m8)

                @pl.when(j >= 4)
                def _():
                    wait_s(m4, (m8 - 4) % 8)

                pltpu.async_copy(h_hbm.at[sbuf.at[m8]], msgs.at[m4],
                                 sem_g.at[m4])

            @pl.when(j >= 3)
            def _():
                pltpu.make_async_copy(h_hbm.at[sbuf.at[md8]],
                                      msgs.at[md4], sem_g.at[md4]).wait()
                for i in range(BATCH // 16):
                    dstbuf[md8, pl.ds(i * 16, 16)] = lbuf[md8,
                                                          pl.ds(i * 16, 16)]

                @pl.when(j + 5 < nb)
                def _():
                    start_e(j + 5, md8)

                pltpu.async_copy(msgs.at[md4], acc.at[dstbuf.at[md8]],
                                 sem_s.at[md4], add=True)

        @pl.loop(0, nb + 3)
        def _(j):
            for m in range(8):
                @pl.when(j % 8 == m)
                def _(m=m):
                    step(j, m)

        for t in range(4):
            g = nb - 4 + t
            wait_s(g % 4, g % 8)

        plsc.subcore_barrier()

        pltpu.sync_copy(acc.at[_slab(s)], a_hbm.at[_slab(s)])

        @pl.when(s == NS - 1)
        def _():
            pltpu.sync_copy(acc.at[_TAIL], a_hbm.at[_TAIL])

    @pl.when(c == 0)
    def _():
        run(h0_hbm, a0_hbm)

    @pl.when(c == 1)
    def _():
        run(h1_hbm, a1_hbm)


def _dinv(p0, p1):
    return lax.rsqrt(1.0 + p0 + p1)


def _l1_body(x_ref, w_ref, p0_ref, p1_ref, h0_ref, h1_ref):
    dinv = _dinv(p0_ref[...], p1_ref[...])
    h = jnp.dot(x_ref[...], w_ref[...], preferred_element_type=jnp.float32)
    h = h * dinv
    h0_ref[...] = h[:, :H]
    h1_ref[...] = h[:, H:]


def _mid_body(a0_ref, a1_ref, p0_ref, p1_ref, b_ref, w_ref, h0_ref, h1_ref):
    dinv = _dinv(p0_ref[...], p1_ref[...])
    agg = jnp.concatenate([a0_ref[...], a1_ref[...]], axis=1)
    u = jnp.maximum(agg * dinv + b_ref[...], 0.0)
    h = jnp.dot(u, w_ref[...], preferred_element_type=jnp.float32)
    h = h * dinv
    h0_ref[...] = h[:, :H]
    h1_ref[...] = h[:, H:]


def _final_body(a0_ref, a1_ref, p0_ref, p1_ref, b_ref, out_ref):
    dinv = _dinv(p0_ref[...], p1_ref[...])
    agg = jnp.concatenate([a0_ref[...], a1_ref[...]], axis=1)
    out_ref[...] = agg * dinv + b_ref[...]


_row_spec = pl.BlockSpec((ROW_BLK, D), lambda i: (i, 0))
_half_spec = pl.BlockSpec((ROW_BLK, H), lambda i: (i, 0))
_p_spec = pl.BlockSpec((ROW_BLK, 1), lambda i: (i, 0))
_w_spec = pl.BlockSpec((D, D), lambda i: (0, 0))
_b_spec = pl.BlockSpec((1, D), lambda i: (0, 0))
_GRID = (N // ROW_BLK,)

_l1_call = pl.pallas_call(
    _l1_body,
    grid=_GRID,
    in_specs=[_row_spec, _w_spec, _p_spec, _p_spec],
    out_specs=[_half_spec, _half_spec],
    out_shape=[
        jax.ShapeDtypeStruct((N, H), jnp.float32),
        jax.ShapeDtypeStruct((N, H), jnp.float32),
    ],
)

_mid_call = pl.pallas_call(
    _mid_body,
    grid=_GRID,
    in_specs=[_half_spec, _half_spec, _p_spec, _p_spec, _b_spec, _w_spec],
    out_specs=[_half_spec, _half_spec],
    out_shape=[
        jax.ShapeDtypeStruct((N, H), jnp.float32),
        jax.ShapeDtypeStruct((N, H), jnp.float32),
    ],
)

_final_call = pl.pallas_call(
    _final_body,
    grid=_GRID,
    in_specs=[_half_spec, _half_spec, _p_spec, _p_spec, _b_spec],
    out_specs=_row_spec,
    out_shape=jax.ShapeDtypeStruct((N, D), jnp.float32),
)


def kernel(x, edge_index, W1, b1, W2, b2, W3, b3):
    ei = edge_index.astype(jnp.int32).reshape(2 * E)
    p0, p1 = _deg_kernel(ei)
    p0 = p0.reshape(N, 1)
    p1 = p1.reshape(N, 1)

    h0, h1 = _l1_call(x, W1, p0, p1)
    a0, a1 = _agg_kernel(h0, h1, ei)

    h0, h1 = _mid_call(a0, a1, p0, p1, b1.reshape(1, D), W2)
    a0, a1 = _agg_kernel(h0, h1, ei)

    h0, h1 = _mid_call(a0, a1, p0, p1, b2.reshape(1, D), W3)
    a0, a1 = _agg_kernel(h0, h1, ei)

    return _final_call(a0, a1, p0, p1, b3.reshape(1, D))


# speedup vs baseline: 21.1961x; 1.0153x over previous
import functools

import jax
import jax.numpy as jnp
from jax import lax
from jax.experimental import pallas as pl
from jax.experimental.pallas import tpu as pltpu
from jax.experimental.pallas import tpu_sc as plsc

N = 10000
E = 160000
D = 256
H = D // 2
NS = 16
BATCH = 80
ROWS_PER_TILE = 624
ROW_BLK = 5000

_mesh = plsc.VectorSubcoreMesh(core_axis_name="core", subcore_axis_name="subcore")


def _slab(s):
    return pl.ds(s * ROWS_PER_TILE, ROWS_PER_TILE)


_TAIL = pl.ds(NS * ROWS_PER_TILE, N - NS * ROWS_PER_TILE)


_EDGES_PER_CORE = E // 2
_DEG_EPT = _EDGES_PER_CORE // NS
_DEG_FULL = _DEG_EPT // BATCH
_DEG_TAILN = _DEG_EPT - _DEG_FULL * BATCH


@functools.partial(
    pl.kernel,
    out_type=[
        jax.ShapeDtypeStruct((N,), jnp.float32),
        jax.ShapeDtypeStruct((N,), jnp.float32),
    ],
    mesh=_mesh,
    scratch_types=[
        pltpu.VMEM_SHARED((N,), jnp.float32),
        pltpu.VMEM((4, BATCH), jnp.int32),
        pltpu.VMEM((1, _DEG_TAILN), jnp.int32),
        pltpu.VMEM((BATCH,), jnp.float32),
        pltpu.VMEM((ROWS_PER_TILE + 16,), jnp.float32),
        pltpu.SemaphoreType.DMA((4,)),
        pltpu.SemaphoreType.DMA((4,)),
    ],
)
def _deg_kernel(ei_hbm, p0_hbm, p1_hbm, deg_sp, dbuf, tbuf, ones, zbuf,
                sem_e, sem_s):
    c = lax.axis_index("core")
    s = lax.axis_index("subcore")

    @pl.loop(0, (ROWS_PER_TILE + 16) // 16)
    def _(i):
        zbuf[pl.ds(i * 16, 16)] = jnp.zeros((16,), jnp.float32)

    @pl.loop(0, BATCH // 16)
    def _(i):
        ones[pl.ds(i * 16, 16)] = jnp.ones((16,), jnp.float32)

    pltpu.sync_copy(zbuf.at[pl.ds(0, ROWS_PER_TILE)], deg_sp.at[_slab(s)])

    @pl.when(s == NS - 1)
    def _():
        pltpu.sync_copy(zbuf.at[pl.ds(0, 16)], deg_sp.at[_TAIL])

    plsc.subcore_barrier()

    tile_base = E + c * _EDGES_PER_CORE + s * _DEG_EPT

    def start_e(j, k):
        pltpu.async_copy(ei_hbm.at[pl.ds(tile_base + j * BATCH, BATCH)],
                         dbuf.at[k], sem_e.at[k])

    def wait_e(k):
        pltpu.make_async_copy(ei_hbm.at[pl.ds(0, BATCH)], dbuf.at[k],
                              sem_e.at[k]).wait()

    def wait_s(k):
        pltpu.make_async_copy(ones, deg_sp.at[dbuf.at[k]], sem_s.at[k]).wait()

    for k in range(4):
        start_e(k, k)

    @pl.loop(0, _DEG_FULL)
    def _(j):
        for m in range(4):
            @pl.when(j % 4 == m)
            def _(m=m):
                wait_e(m)

                @pl.when(j >= 4)
                def _():
                    wait_s(m)

                pltpu.async_copy(ones, deg_sp.at[dbuf.at[m]], sem_s.at[m],
                                 add=True)

                @pl.when(j + 4 < _DEG_FULL)
                def _():
                    start_e(j + 4, m)

    pltpu.sync_copy(ei_hbm.at[pl.ds(tile_base + _DEG_FULL * BATCH, _DEG_TAILN)],
                    tbuf.at[0])
    pltpu.sync_copy(ones.at[pl.ds(0, _DEG_TAILN)], deg_sp.at[tbuf.at[0]],
                    add=True)

    for k in range(4):
        wait_s(k)

    plsc.subcore_barrier()

    @pl.when(jnp.logical_and(s == 0, c == 0))
    def _():
        pltpu.sync_copy(deg_sp, p0_hbm)

    @pl.when(jnp.logical_and(s == 0, c == 1))
    def _():
        pltpu.sync_copy(deg_sp, p1_hbm)


_AGG_BATCHES = E // BATCH
_AGG_BPT = _AGG_BATCHES // NS
assert _AGG_BPT * NS == _AGG_BATCHES


@functools.partial(
    pl.kernel,
    out_type=[
        jax.ShapeDtypeStruct((N, H), jnp.float32),
        jax.ShapeDtypeStruct((N, H), jnp.float32),
    ],
    mesh=_mesh,
    scratch_types=[
        pltpu.VMEM_SHARED((N, H), jnp.float32),
        pltpu.VMEM((8, BATCH), jnp.int32),
        pltpu.VMEM((8, BATCH), jnp.int32),
        pltpu.VMEM((8, BATCH), jnp.int32),
        pltpu.VMEM((4, BATCH, H), jnp.float32),
        pltpu.SemaphoreType.DMA((8,)),
        pltpu.SemaphoreType.DMA((4,)),
        pltpu.SemaphoreType.DMA((4,)),
    ],
)
def _agg_kernel(h0_hbm, h1_hbm, ei_hbm, a0_hbm, a1_hbm, acc, sbuf, lbuf,
                dstbuf, msgs, sem_e, sem_g, sem_s):
    c = lax.axis_index("core")
    s = lax.axis_index("subcore")
    nb = _AGG_BPT

    def run(h_hbm, a_hbm):
        pltpu.sync_copy(h_hbm.at[_slab(s)], acc.at[_slab(s)])

        @pl.when(s == NS - 1)
        def _():
            pltpu.sync_copy(h_hbm.at[_TAIL], acc.at[_TAIL])

        plsc.subcore_barrier()

        def start_e(j, k):
            base = (j * NS + s) * BATCH
            pltpu.async_copy(ei_hbm.at[pl.ds(base, BATCH)], sbuf.at[k],
                             sem_e.at[k])
            pltpu.async_copy(ei_hbm.at[pl.ds(E + base, BATCH)], lbuf.at[k],
                             sem_e.at[k])

        def wait_e(k):
            pltpu.make_async_copy(ei_hbm.at[pl.ds(0, BATCH)], sbuf.at[k],
                                  sem_e.at[k]).wait()
            pltpu.make_async_copy(ei_hbm.at[pl.ds(0, BATCH)], lbuf.at[k],
                                  sem_e.at[k]).wait()

        def wait_s(k4, k8):
            pltpu.make_async_copy(msgs.at[k4], acc.at[dstbuf.at[k8]],
                                  sem_s.at[k4]).wait()

        for k in range(8):
            start_e(k, k)

        def step(j, m8):
            m4 = m8 % 4
            md8 = (m8 - 3) % 8
            md4 = (m8 - 3) % 4
            @pl.when(j < nb)
            def _():
                wait_e(m8)

                @pl.when(j >= 4)
                def _():
                    wait_s(m4, (m8 - 4) % 8)

                pltpu.async_copy(h_hbm.at[sbuf.at[m8]], msgs.at[m4],
                                 sem_g.at[m4])

            @pl.when(j >= 3)
            def _():
                pltpu.make_async_copy(h_hbm.at[sbuf.at[md8]],
                                      msgs.at[md4], sem_g.at[md4]).wait()
                for i in range(BATCH // 16):
                    dstbuf[md8, pl.ds(i * 16, 16)] = lbuf[md8,
                                                          pl.ds(i * 16, 16)]

                @pl.when(j + 5 < nb)
                def _():
                    start_e(j + 5, md8)

                pltpu.async_copy(msgs.at[md4], acc.at[dstbuf.at[md8]],
                                 sem_s.at[md4], add=True)

        @pl.loop(0, nb + 3)
        def _(j):
            for m in range(8):
                @pl.when(j % 8 == m)
                def _(m=m):
                    step(j, m)

        for t in range(4):
            g = nb - 4 + t
            wait_s(g % 4, g % 8)

        plsc.subcore_barrier()

        pltpu.sync_copy(acc.at[_slab(s)], a_hbm.at[_slab(s)])

        @pl.when(s == NS - 1)
        def _():
            pltpu.sync_copy(acc.at[_TAIL], a_hbm.at[_TAIL])

    @pl.when(c == 0)
    def _():
        run(h0_hbm, a0_hbm)

    @pl.when(c == 1)
    def _():
        run(h1_hbm, a1_hbm)


def _dinv(p0, p1):
    return lax.rsqrt(1.0 + p0 + p1)


def _l1_body(x_ref, w_ref, p0_ref, p1_ref, h0_ref, h1_ref):
    dinv = _dinv(p0_ref[...], p1_ref[...])
    h = jnp.dot(x_ref[...], w_ref[...], preferred_element_type=jnp.float32)
    h = h * dinv
    h0_ref[...] = h[:, :H]
    h1_ref[...] = h[:, H:]


def _mid_body(a0_ref, a1_ref, p0_ref, p1_ref, b_ref, w_ref, h0_ref, h1_ref):
    dinv = _dinv(p0_ref[...], p1_ref[...])
    agg = jnp.concatenate([a0_ref[...], a1_ref[...]], axis=1)
    u = jnp.maximum(agg * dinv + b_ref[...], 0.0)
    h = jnp.dot(u, w_ref[...], preferred_element_type=jnp.float32)
    h = h * dinv
    h0_ref[...] = h[:, :H]
    h1_ref[...] = h[:, H:]


def _final_body(a0_ref, a1_ref, p0_ref, p1_ref, b_ref, out_ref):
    dinv = _dinv(p0_ref[...], p1_ref[...])
    agg = jnp.concatenate([a0_ref[...], a1_ref[...]], axis=1)
    out_ref[...] = agg * dinv + b_ref[...]


_row_spec = pl.BlockSpec((ROW_BLK, D), lambda i: (i, 0))
_half_spec = pl.BlockSpec((ROW_BLK, H), lambda i: (i, 0))
_p_spec = pl.BlockSpec((ROW_BLK, 1), lambda i: (i, 0))
_w_spec = pl.BlockSpec((D, D), lambda i: (0, 0))
_b_spec = pl.BlockSpec((1, D), lambda i: (0, 0))
_GRID = (N // ROW_BLK,)

_l1_call = pl.pallas_call(
    _l1_body,
    grid=_GRID,
    in_specs=[_row_spec, _w_spec, _p_spec, _p_spec],
    out_specs=[_half_spec, _half_spec],
    out_shape=[
        jax.ShapeDtypeStruct((N, H), jnp.float32),
        jax.ShapeDtypeStruct((N, H), jnp.float32),
    ],
)

_mid_call = pl.pallas_call(
    _mid_body,
    grid=_GRID,
    in_specs=[_half_spec, _half_spec, _p_spec, _p_spec, _b_spec, _w_spec],
    out_specs=[_half_spec, _half_spec],
    out_shape=[
        jax.ShapeDtypeStruct((N, H), jnp.float32),
        jax.ShapeDtypeStruct((N, H), jnp.float32),
    ],
)

_final_call = pl.pallas_call(
    _final_body,
    grid=_GRID,
    in_specs=[_half_spec, _half_spec, _p_spec, _p_spec, _b_spec],
    out_specs=_row_spec,
    out_shape=jax.ShapeDtypeStruct((N, D), jnp.float32),
)


def kernel(x, edge_index, W1, b1, W2, b2, W3, b3):
    ei = edge_index.astype(jnp.int32).reshape(2 * E)
    p0, p1 = _deg_kernel(ei)
    p0 = p0.reshape(N, 1)
    p1 = p1.reshape(N, 1)

    h0, h1 = _l1_call(x, W1, p0, p1)
    a0, a1 = _agg_kernel(h0, h1, ei)

    h0, h1 = _mid_call(a0, a1, p0, p1, b1.reshape(1, D), W2)
    a0, a1 = _agg_kernel(h0, h1, ei)

    h0, h1 = _mid_call(a0, a1, p0, p1, b2.reshape(1, D), W3)
    a0, a1 = _agg_kernel(h0, h1, ei)

    return _final_call(a0, a1, p0, p1, b3.reshape(1, D))
